# Initial kernel scaffold; baseline (speedup 1.0000x reference)
import jax
import jax.numpy as jnp
from jax.experimental import pallas as pl


def kernel(x, edge_index, W1, b1, W2, b2):
    raise NotImplementedError("write your pallas kernel here")


# speedup vs baseline: 3.3461x; 3.3461x over previous
import functools

import jax
import jax.numpy as jnp
from jax import lax
from jax.experimental import pallas as pl
from jax.experimental.pallas import tpu as pltpu
from jax.experimental.pallas import tpu_sc as plsc


def _make_sc_agg(n_pad: int, e_pad: int, d: int, nc: int, ns: int,
                 nchunk: int, ch: int):
    nw = nc * ns
    rows_per_tile = n_pad // ns
    mesh = plsc.VectorSubcoreMesh(core_axis_name="c", subcore_axis_name="s")

    @functools.partial(
        pl.kernel,
        mesh=mesh,
        out_type=jax.ShapeDtypeStruct((nc, n_pad, d), jnp.float32),
        scratch_types=[
            pltpu.VMEM((nchunk, ch), jnp.int32),
            pltpu.VMEM((nchunk, ch), jnp.int32),
            pltpu.VMEM((ch, d), jnp.float32),
            pltpu.VMEM_SHARED((n_pad, d), jnp.float32),
            pltpu.SemaphoreType.DMA,
        ],
    )
    def sc_agg(x_hbm, src_hbm, dst_hbm, zeros_hbm, out_hbm,
               src_v, dst_v, rows0, acc, sem):
        c = lax.axis_index("c")
        s = lax.axis_index("s")
        wid = s * nc + c
        r0 = s * rows_per_tile
        pltpu.sync_copy(zeros_hbm.at[pl.ds(r0, rows_per_tile)],
                        acc.at[pl.ds(r0, rows_per_tile)])
        plsc.subcore_barrier()

        pltpu.sync_copy(src_hbm.at[wid], src_v)
        pltpu.sync_copy(dst_hbm.at[wid], dst_v)

        def body(j, carry):
            pltpu.async_copy(x_hbm.at[src_v.at[j]], rows0, sem).wait()
            pltpu.sync_copy(rows0, acc.at[dst_v.at[j]], add=True)
            return carry

        lax.fori_loop(0, nchunk, body, 0)

        plsc.subcore_barrier()
        pltpu.sync_copy(acc.at[pl.ds(r0, rows_per_tile)],
                        out_hbm.at[c, pl.ds(r0, rows_per_tile)])

    return sc_agg


def _mlp_body(x_ref, p0_ref, p1_ref, w1_ref, b1_ref, w2_ref, b2_ref, o_ref):
    h = x_ref[...] + p0_ref[...] + p1_ref[...]
    h = jnp.maximum(
        jnp.dot(h, w1_ref[...], preferred_element_type=jnp.float32)
        + b1_ref[...], 0.0)
    o_ref[...] = (
        jnp.dot(h, w2_ref[...], preferred_element_type=jnp.float32)
        + b2_ref[...])


def kernel(x, edge_index, W1, b1, W2, b2):
    n, d = x.shape
    e = edge_index.shape[1]

    info = plsc.get_sparse_core_info()
    nc, ns = info.num_cores, info.num_subcores
    nw = nc * ns

    ch = 128
    epw = -(-e // (nw * 2 * ch)) * 2 * ch
    nchunk = epw // ch
    e_pad = nw * epw
    n_pad = -(-n // (ns * 8)) * (ns * 8)

    src = edge_index[0]
    dst = edge_index[1]
    pad = e_pad - e
    if pad:
        src = jnp.concatenate([src, jnp.zeros((pad,), jnp.int32)])
        dst = jnp.concatenate([dst, jnp.full((pad,), n, jnp.int32)])
    src3 = src.reshape(nw, nchunk, ch)
    dst3 = dst.reshape(nw, nchunk, ch)
    zeros = jnp.zeros((n_pad, d), jnp.float32)

    sc_agg = _make_sc_agg(n_pad, e_pad, d, nc, ns, nchunk, ch)
    parts = sc_agg(x, src3, dst3, zeros)

    blk = 400
    grid = (-(-n // blk),)
    out = pl.pallas_call(
        _mlp_body,
        grid=grid,
        in_specs=[
            pl.BlockSpec((blk, d), lambda i: (i, 0)),
            pl.BlockSpec((blk, d), lambda i: (i, 0)),
            pl.BlockSpec((blk, d), lambda i: (i, 0)),
            pl.BlockSpec((d, d), lambda i: (0, 0)),
            pl.BlockSpec((1, d), lambda i: (0, 0)),
            pl.BlockSpec((d, d), lambda i: (0, 0)),
            pl.BlockSpec((1, d), lambda i: (0, 0)),
        ],
        out_specs=pl.BlockSpec((blk, d), lambda i: (i, 0)),
        out_shape=jax.ShapeDtypeStruct((n, d), jnp.float32),
    )(x, parts[0, :n], parts[1, :n], W1, b1.reshape(1, d), W2,
      b2.reshape(1, d))
    return out


# speedup vs baseline: 3.6990x; 1.1055x over previous
import functools

import jax
import jax.numpy as jnp
from jax import lax
from jax.experimental import pallas as pl
from jax.experimental.pallas import tpu as pltpu
from jax.experimental.pallas import tpu_sc as plsc


def _make_sc_agg(n_pad: int, d: int, nc: int, ns: int,
                 nsup: int, g: int, ch: int):
    nw = nc * ns
    rows_per_tile = n_pad // ns
    mesh = plsc.VectorSubcoreMesh(core_axis_name="c", subcore_axis_name="s")

    @functools.partial(
        pl.kernel,
        mesh=mesh,
        out_type=jax.ShapeDtypeStruct((nc, n_pad, d), jnp.float32),
        scratch_types=[
            pltpu.VMEM((g, ch), jnp.int32),
            pltpu.VMEM((g, ch), jnp.int32),
            pltpu.VMEM((g, ch), jnp.int32),
            pltpu.VMEM((g, ch), jnp.int32),
            pltpu.VMEM((ch, d), jnp.float32),
            pltpu.VMEM((ch, d), jnp.float32),
            pltpu.VMEM_SHARED((n_pad, d), jnp.float32),
            pltpu.SemaphoreType.DMA,
            pltpu.SemaphoreType.DMA,
            pltpu.SemaphoreType.DMA,
        ],
    )
    def sc_agg(x_hbm, src_hbm, dst_hbm, zeros_hbm, out_hbm,
               sw0, sw1, dw0, dw1, rows0, rows1, acc, sem_g, sem_s, sem_i):
        c = lax.axis_index("c")
        s = lax.axis_index("s")
        wid = s * nc + c
        r0 = s * rows_per_tile
        pltpu.sync_copy(zeros_hbm.at[pl.ds(r0, rows_per_tile)],
                        acc.at[pl.ds(r0, rows_per_tile)])
        plsc.subcore_barrier()

        sw = (sw0, sw1)
        dw = (dw0, dw1)
        rows = (rows0, rows1)

        pltpu.sync_copy(src_hbm.at[wid, 0], sw[0])
        pltpu.sync_copy(dst_hbm.at[wid, 0], dw[0])

        for k in range(nsup):
            w = k % 2
            if k > 0:
                pltpu.make_async_copy(src_hbm.at[wid, k], sw[w],
                                      sem_i).wait()
                pltpu.make_async_copy(dst_hbm.at[wid, k], dw[w],
                                      sem_i).wait()
            if k + 1 < nsup:
                pltpu.async_copy(src_hbm.at[wid, k + 1], sw[1 - w], sem_i)
                pltpu.async_copy(dst_hbm.at[wid, k + 1], dw[1 - w], sem_i)
            swin, dwin = sw[w], dw[w]

            def start_g(j, b):
                pltpu.async_copy(x_hbm.at[swin.at[j]], rows[b], sem_g)

            def wait_g(j, b):
                pltpu.make_async_copy(x_hbm.at[swin.at[j]], rows[b],
                                      sem_g).wait()

            def start_s(j, b):
                pltpu.async_copy(rows[b], acc.at[dwin.at[j]], sem_s,
                                 add=True)

            def wait_s(j, b):
                pltpu.make_async_copy(rows[b], acc.at[dwin.at[j]],
                                      sem_s).wait()

            start_g(0, 0)
            start_g(1, 1)
            wait_g(0, 0)
            start_s(0, 0)

            def body(t, carry):
                j1 = 2 * t + 1
                wait_s(j1 - 1, 0)
                start_g(j1 + 1, 0)
                wait_g(j1, 1)
                start_s(j1, 1)
                j2 = j1 + 1
                wait_s(j2 - 1, 1)
                start_g(j2 + 1, 1)
                wait_g(j2, 0)
                start_s(j2, 0)
                return carry

            lax.fori_loop(0, (g - 2) // 2, body, 0)
            jl = g - 1
            wait_s(jl - 1, 0)
            wait_g(jl, 1)
            start_s(jl, 1)
            wait_s(jl, 1)

        plsc.subcore_barrier()
        pltpu.sync_copy(acc.at[pl.ds(r0, rows_per_tile)],
                        out_hbm.at[c, pl.ds(r0, rows_per_tile)])

    return sc_agg


def _mlp_body(x_ref, p0_ref, p1_ref, w1_ref, b1_ref, w2_ref, b2_ref, o_ref):
    h = x_ref[...] + p0_ref[...] + p1_ref[...]
    h = jnp.maximum(
        jnp.dot(h, w1_ref[...], preferred_element_type=jnp.float32)
        + b1_ref[...], 0.0)
    o_ref[...] = (
        jnp.dot(h, w2_ref[...], preferred_element_type=jnp.float32)
        + b2_ref[...])


def kernel(x, edge_index, W1, b1, W2, b2):
    n, d = x.shape
    e = edge_index.shape[1]

    info = plsc.get_sparse_core_info()
    nc, ns = info.num_cores, info.num_subcores
    nw = nc * ns

    ch = 128
    g = 16
    nsup = -(-e // (nw * g * ch))
    epw = nsup * g * ch
    e_pad = nw * epw
    n_pad = -(-n // (ns * 8)) * (ns * 8)

    src = edge_index[0]
    dst = edge_index[1]
    pad = e_pad - e
    if pad:
        src = jnp.concatenate([src, jnp.zeros((pad,), jnp.int32)])
        dst = jnp.concatenate([dst, jnp.full((pad,), n, jnp.int32)])
    src4 = src.reshape(nw, nsup, g, ch)
    dst4 = dst.reshape(nw, nsup, g, ch)
    zeros = jnp.zeros((n_pad, d), jnp.float32)

    sc_agg = _make_sc_agg(n_pad, d, nc, ns, nsup, g, ch)
    parts = sc_agg(x, src4, dst4, zeros)

    blk = 400
    grid = (-(-n // blk),)
    out = pl.pallas_call(
        _mlp_body,
        grid=grid,
        in_specs=[
            pl.BlockSpec((blk, d), lambda i: (i, 0)),
            pl.BlockSpec((blk, d), lambda i: (i, 0)),
            pl.BlockSpec((blk, d), lambda i: (i, 0)),
            pl.BlockSpec((d, d), lambda i: (0, 0)),
            pl.BlockSpec((1, d), lambda i: (0, 0)),
            pl.BlockSpec((d, d), lambda i: (0, 0)),
            pl.BlockSpec((1, d), lambda i: (0, 0)),
        ],
        out_specs=pl.BlockSpec((blk, d), lambda i: (i, 0)),
        out_shape=jax.ShapeDtypeStruct((n, d), jnp.float32),
    )(x, parts[0, :n], parts[1, :n], W1, b1.reshape(1, d), W2,
      b2.reshape(1, d))
    return out
